# baseline (speedup 1.0000x reference)
import functools

import jax
import jax.numpy as jnp
import numpy as np
from jax import lax
from jax.experimental import pallas as pl
from jax.experimental.pallas import tpu as pltpu
from jax.experimental.pallas import tpu_sc as plsc

N = 1000000
D = 16
NC = 2
NS = 16
NW = NC * NS
L = 16

NP = 1 << 20
SEG = NP // NW
VSEG = SEG // L
HSEG = SEG // 2
HVSEG = HSEG // L

BITS = (11, 11, 10)
SHIFTS = (0, 11, 22)
SIGN = np.int32(-2147483648)

_MESH = plsc.VectorSubcoreMesh(core_axis_name="c", subcore_axis_name="s",
                               num_cores=NC, num_subcores=NS)
_SC_PARAMS = pltpu.CompilerParams(use_tc_tiling_on_sc=False,
                                  needs_layout_passes=False)


def _wid():
    return lax.axis_index("s") * NC + lax.axis_index("c")


def _to_key(b):
    return jnp.where(b < 0, ~b, b | SIGN)


def _from_key(k):
    return plsc.bitcast(jnp.where(k < 0, k ^ SIGN, ~k), jnp.float32)


def _digit(k, shift, mask):
    return lax.shift_right_logical(k, np.int32(shift)) & np.int32(mask)


def _make_hist(shift, nbits, first_pass):
    r = 1 << nbits
    mask = r - 1
    stage = (pltpu.VMEM((SEG,), jnp.int32) if first_pass
             else pltpu.VMEM((2 * SEG,), jnp.int32))

    @functools.partial(
        pl.kernel,
        out_type=jax.ShapeDtypeStruct((NW * r,), jnp.int32),
        mesh=_MESH,
        scratch_types=(
            stage,
            pltpu.VMEM((r,), jnp.int32),
            pltpu.SemaphoreType.DMA,
        ),
        compiler_params=_SC_PARAMS,
    )
    def hist_kernel(key_hbm, hist_hbm, key_v, hist_v, sem):
        w = _wid()
        zeros = lax.full((L,), np.int32(0), jnp.int32)
        for i in range(r // L):
            hist_v[pl.ds(i * L, L)] = zeros
        if first_pass:
            pltpu.sync_copy(key_hbm.at[pl.ds(w * SEG, SEG)], key_v)
        else:
            pltpu.sync_copy(key_hbm.at[pl.ds(w * 2 * SEG, 2 * SEG)], key_v)
        dl16 = lax.iota(jnp.int32, L)

        def body(i, _):
            if first_pass:
                k = _to_key(key_v[pl.ds(i * L, L)])
            else:
                k = plsc.load_gather(key_v, [(i * L + dl16) * 2])
            d = _digit(k, shift, mask)
            cnt, last = plsc.scan_count(d)
            plsc.addupdate_scatter(hist_v, [d], cnt, mask=last)
            return 0

        lax.fori_loop(0, VSEG, body, 0)
        pltpu.sync_copy(hist_v, hist_hbm.at[pl.ds(w * r, r)])

    return hist_kernel


def _make_scan(nbits):
    r = 1 << nbits
    dpw = r // NW

    @functools.partial(
        pl.kernel,
        out_type=(
            jax.ShapeDtypeStruct((r * NW,), jnp.int32),
            jax.ShapeDtypeStruct((r,), jnp.int32),
        ),
        mesh=_MESH,
        scratch_types=(
            pltpu.VMEM((NW * dpw,), jnp.int32),
            pltpu.VMEM((dpw * NW,), jnp.int32),
            pltpu.VMEM((dpw,), jnp.int32),
            pltpu.SemaphoreType.DMA,
        ),
        compiler_params=_SC_PARAMS,
    )
    def scan_kernel(hist_hbm, prefix_hbm, tot_hbm, rows_v, pref_v, tot_v, sem):
        w = _wid()
        for wp in range(NW):
            pltpu.sync_copy(
                hist_hbm.at[pl.ds(wp * r + w * dpw, dpw)],
                rows_v.at[pl.ds(wp * dpw, dpw)],
            )
        dl16 = lax.iota(jnp.int32, L)

        def dblock(i, _):
            dbase = i * L
            acc0 = lax.full((L,), np.int32(0), jnp.int32)

            def wloop(wp, acc):
                c = plsc.load_gather(rows_v, [wp * dpw + dbase + dl16])
                plsc.store_scatter(pref_v, [(dbase + dl16) * NW + wp], acc)
                return acc + c

            acc = lax.fori_loop(0, NW, wloop, acc0)
            tot_v[pl.ds(dbase, L)] = acc
            return 0

        lax.fori_loop(0, dpw // L, dblock, 0)
        pltpu.sync_copy(pref_v, prefix_hbm.at[pl.ds(w * dpw * NW, dpw * NW)])
        pltpu.sync_copy(tot_v, tot_hbm.at[pl.ds(w * dpw, dpw)])

    return scan_kernel


def _make_permute(shift, nbits, first_pass):
    r = 1 << nbits
    mask = r - 1
    if first_pass:
        stage = pltpu.VMEM((SEG,), jnp.int32)
        comb = pltpu.VMEM((2 * HSEG,), jnp.int32)
    else:
        stage = pltpu.VMEM((2 * SEG,), jnp.int32)
        comb = None

    scratch = [
        stage,
        pltpu.VMEM((2 * HSEG,), jnp.int32),
        pltpu.VMEM((r,), jnp.int32),
        pltpu.VMEM((r,), jnp.int32),
        pltpu.VMEM((r,), jnp.int32),
        pltpu.SemaphoreType.DMA,
    ]
    if comb is not None:
        scratch.insert(1, comb)
    out_type = jax.ShapeDtypeStruct((2 * NP,), jnp.int32)

    def setup(w, prefix_hbm, tot_hbm, tot_v, col_v, ctr_v, sem):
        pltpu.sync_copy(tot_hbm, tot_v)

        def mkidx(i, _):
            ctr_v[pl.ds(i * L, L)] = (i * L + lax.iota(jnp.int32, L)) * NW + w
            return 0
        lax.fori_loop(0, r // L, mkidx, 0)
        pltpu.async_copy(prefix_hbm.at[ctr_v], col_v, sem).wait()

        def scan_step(i, carry):
            t = tot_v[pl.ds(i * L, L)]
            inc = plsc.cumsum(t)
            ctr_v[pl.ds(i * L, L)] = inc - t + carry + col_v[pl.ds(i * L, L)]
            return carry + jnp.sum(t)
        lax.fori_loop(0, r // L, scan_step, np.int32(0))

    if first_pass:
        @functools.partial(
            pl.kernel, out_type=out_type, mesh=_MESH,
            scratch_types=tuple(scratch), compiler_params=_SC_PARAMS,
        )
        def permute_kernel(key_hbm, prefix_hbm, tot_hbm, pout_hbm,
                           key_v, comb_v, cidx_v, tot_v, col_v, ctr_v, sem):
            w = _wid()
            pltpu.sync_copy(key_hbm.at[pl.ds(w * SEG, SEG)], key_v)
            setup(w, prefix_hbm, tot_hbm, tot_v, col_v, ctr_v, sem)
            dl16 = lax.iota(jnp.int32, L)

            for h in range(2):
                def body(i, _):
                    j16 = i * L + dl16
                    g16 = h * HSEG + j16
                    k = _to_key(key_v[pl.ds(h * HSEG + i * L, L)])
                    d = _digit(k, shift, mask)
                    cnt, last = plsc.scan_count(d)
                    cur = plsc.load_gather(ctr_v, [d])
                    pos2 = (cur + cnt - 1) * 2
                    plsc.store_scatter(comb_v, [j16 * 2], k)
                    plsc.store_scatter(comb_v, [j16 * 2 + 1], w * SEG + g16)
                    plsc.store_scatter(cidx_v, [j16 * 2], pos2)
                    plsc.store_scatter(cidx_v, [j16 * 2 + 1], pos2 + 1)
                    plsc.addupdate_scatter(ctr_v, [d], cnt, mask=last)
                    return 0
                lax.fori_loop(0, HVSEG, body, 0)
                pltpu.async_copy(comb_v, pout_hbm.at[cidx_v], sem).wait()
    else:
        @functools.partial(
            pl.kernel, out_type=out_type, mesh=_MESH,
            scratch_types=tuple(scratch), compiler_params=_SC_PARAMS,
        )
        def permute_kernel(pair_hbm, prefix_hbm, tot_hbm, pout_hbm,
                           pair_v, cidx_v, tot_v, col_v, ctr_v, sem):
            w = _wid()
            pltpu.sync_copy(pair_hbm.at[pl.ds(w * 2 * SEG, 2 * SEG)], pair_v)
            setup(w, prefix_hbm, tot_hbm, tot_v, col_v, ctr_v, sem)
            dl16 = lax.iota(jnp.int32, L)

            for h in range(2):
                def body(i, _):
                    j16 = i * L + dl16
                    k = plsc.load_gather(pair_v,
                                         [(h * HSEG + i * L + dl16) * 2])
                    d = _digit(k, shift, mask)
                    cnt, last = plsc.scan_count(d)
                    cur = plsc.load_gather(ctr_v, [d])
                    pos2 = (cur + cnt - 1) * 2
                    plsc.store_scatter(cidx_v, [j16 * 2], pos2)
                    plsc.store_scatter(cidx_v, [j16 * 2 + 1], pos2 + 1)
                    plsc.addupdate_scatter(ctr_v, [d], cnt, mask=last)
                    return 0
                lax.fori_loop(0, HVSEG, body, 0)
                pltpu.async_copy(pair_v.at[pl.ds(h * 2 * HSEG, 2 * HSEG)],
                                 pout_hbm.at[cidx_v], sem).wait()

    return permute_kernel


CHUNK = 2048
NFULL = N // CHUNK
TAIL = N - NFULL * CHUNK
JMAX = (NFULL + NW) // NW


@functools.partial(
    pl.kernel,
    out_type=(
        jax.ShapeDtypeStruct((N, D), jnp.float32),
        jax.ShapeDtypeStruct((N,), jnp.float32),
    ),
    mesh=_MESH,
    scratch_types=(
        pltpu.VMEM((2 * CHUNK,), jnp.int32),
        pltpu.VMEM((CHUNK,), jnp.int32),
        pltpu.VMEM((CHUNK, D), jnp.float32),
        pltpu.VMEM((CHUNK,), jnp.float32),
        pltpu.SemaphoreType.DMA,
    ),
    compiler_params=_SC_PARAMS,
)
def _gather_kernel(x_hbm, pair_hbm, xs_hbm, fs_hbm,
                   pair_v, idx_v, rows_v, fit_v, sem):
    w = _wid()
    dl16 = lax.iota(jnp.int32, L)
    for j in range(JMAX):
        c = w + j * NW
        off = c * CHUNK

        def do(n):
            pltpu.sync_copy(pair_hbm.at[pl.ds(off * 2, n * 2)],
                            pair_v.at[pl.ds(0, n * 2)])

            def split(i, _):
                j16 = i * L + dl16
                k = plsc.load_gather(pair_v, [j16 * 2])
                ix = plsc.load_gather(pair_v, [j16 * 2 + 1])
                fit_v[pl.ds(i * L, L)] = _from_key(k)
                idx_v[pl.ds(i * L, L)] = ix
                return 0
            lax.fori_loop(0, n // L, split, 0)

            pltpu.async_copy(x_hbm.at[idx_v.at[pl.ds(0, n)]],
                             rows_v.at[pl.ds(0, n)], sem).wait()
            pltpu.sync_copy(rows_v.at[pl.ds(0, n)],
                            xs_hbm.at[pl.ds(off, n)])
            pltpu.sync_copy(fit_v.at[pl.ds(0, n)],
                            fs_hbm.at[pl.ds(off, n)])

        @pl.when(c < NFULL)
        def _full():
            do(CHUNK)

        @pl.when(c == NFULL)
        def _tail():
            do(TAIL)


def kernel(x, fitness):
    bits = lax.bitcast_convert_type(fitness, jnp.int32)
    pad = lax.full((NP - N,), np.int32(0x7F800000), jnp.int32)
    k = jnp.concatenate([bits, pad])
    for p in range(3):
        h = _make_hist(SHIFTS[p], BITS[p], p == 0)(k)
        pref, tot = _make_scan(BITS[p])(h)
        k = _make_permute(SHIFTS[p], BITS[p], p == 0)(k, pref, tot)
    x_sorted, fitness_sorted = _gather_kernel(x, k)
    return (x_sorted, fitness_sorted)


# speedup vs baseline: 6.0967x; 6.0967x over previous
import functools

import jax
import jax.numpy as jnp
import numpy as np
from jax import lax
from jax.experimental import pallas as pl
from jax.experimental.pallas import tpu as pltpu
from jax.experimental.pallas import tpu_sc as plsc

N = 1000000
D = 16
NC = 2
NS = 16
NW = NC * NS
L = 16

NP = 1000448
SEG = NP // NW
VSEG = SEG // L
HSEG = SEG // 2
HVSEG = HSEG // L
ZONE = NP // NS

BITS = (11, 11, 10)
SHIFTS = (0, 11, 22)
SIGN = np.int32(-2147483648)

_MESH = plsc.VectorSubcoreMesh(core_axis_name="c", subcore_axis_name="s",
                               num_cores=NC, num_subcores=NS)
_SC_PARAMS = pltpu.CompilerParams(use_tc_tiling_on_sc=False,
                                  needs_layout_passes=False)


def _wid():
    return lax.axis_index("s") * NC + lax.axis_index("c")


def _to_key(b):
    return jnp.where(b < 0, ~b, b | SIGN)


def _from_key(k):
    return plsc.bitcast(jnp.where(k < 0, k ^ SIGN, ~k), jnp.float32)


def _digit(k, shift, mask):
    return lax.shift_right_logical(k, np.int32(shift)) & np.int32(mask)


def _zero16():
    return lax.full((L,), np.int32(0), jnp.int32)


def _make_hist(shift, nbits, first_pass):
    r = 1 << nbits
    mask = r - 1

    @functools.partial(
        pl.kernel,
        out_type=jax.ShapeDtypeStruct((NW * r,), jnp.int32),
        mesh=_MESH,
        scratch_types=(
            pltpu.VMEM((SEG,), jnp.int32),
            pltpu.VMEM((r,), jnp.int32),
            pltpu.SemaphoreType.DMA,
        ),
        compiler_params=_SC_PARAMS,
    )
    def hist_kernel(key_hbm, hist_hbm, key_v, hist_v, sem):
        w = _wid()
        for i in range(r // L):
            hist_v[pl.ds(i * L, L)] = _zero16()
        pltpu.sync_copy(key_hbm.at[pl.ds(w * SEG, SEG)], key_v)

        def body(i, _):
            k = key_v[pl.ds(i * L, L)]
            if first_pass:
                k = _to_key(k)
            d = _digit(k, shift, mask)
            cnt, last = plsc.scan_count(d)
            plsc.addupdate_scatter(hist_v, [d], cnt, mask=last)
            return 0

        lax.fori_loop(0, VSEG, body, 0)
        pltpu.sync_copy(hist_v, hist_hbm.at[pl.ds(w * r, r)])

    return hist_kernel


def _make_scan(nbits):
    r = 1 << nbits
    dpw = r // NW

    @functools.partial(
        pl.kernel,
        out_type=(
            jax.ShapeDtypeStruct((r * NW,), jnp.int32),
            jax.ShapeDtypeStruct((r,), jnp.int32),
        ),
        mesh=_MESH,
        scratch_types=(
            pltpu.VMEM((NW * dpw,), jnp.int32),
            pltpu.VMEM((dpw * NW,), jnp.int32),
            pltpu.VMEM((dpw,), jnp.int32),
            pltpu.SemaphoreType.DMA,
        ),
        compiler_params=_SC_PARAMS,
    )
    def scan_kernel(hist_hbm, prefix_hbm, tot_hbm, rows_v, pref_v, tot_v, sem):
        w = _wid()
        for wp in range(NW):
            pltpu.sync_copy(
                hist_hbm.at[pl.ds(wp * r + w * dpw, dpw)],
                rows_v.at[pl.ds(wp * dpw, dpw)],
            )
        dl16 = lax.iota(jnp.int32, L)

        def dblock(i, _):
            dbase = i * L
            acc0 = _zero16()

            def wloop(wp, acc):
                c = plsc.load_gather(rows_v, [wp * dpw + dbase + dl16])
                plsc.store_scatter(pref_v, [(dbase + dl16) * NW + wp], acc)
                return acc + c

            acc = lax.fori_loop(0, NW, wloop, acc0)
            tot_v[pl.ds(dbase, L)] = acc
            return 0

        lax.fori_loop(0, dpw // L, dblock, 0)
        pltpu.sync_copy(pref_v, prefix_hbm.at[pl.ds(w * dpw * NW, dpw * NW)])
        pltpu.sync_copy(tot_v, tot_hbm.at[pl.ds(w * dpw, dpw)])

    return scan_kernel


def _make_permute(shift, nbits, first_pass):
    r = 1 << nbits
    mask = r - 1

    scratch = (
        pltpu.VMEM((HSEG,), jnp.int32),
        pltpu.VMEM((HSEG,), jnp.int32),
        pltpu.VMEM((HSEG,), jnp.int32),
        pltpu.VMEM((r,), jnp.int32),
        pltpu.VMEM((r,), jnp.int32),
        pltpu.VMEM((r,), jnp.int32),
        pltpu.VMEM_SHARED((NP,), jnp.int32),
        pltpu.SemaphoreType.DMA,
    )
    out_type = (
        jax.ShapeDtypeStruct((NP,), jnp.int32),
        jax.ShapeDtypeStruct((NP,), jnp.int32),
        jax.ShapeDtypeStruct((NP,), jnp.int32),
        jax.ShapeDtypeStruct((NP,), jnp.int32),
    )

    @functools.partial(
        pl.kernel, out_type=out_type, mesh=_MESH,
        scratch_types=scratch, compiler_params=_SC_PARAMS,
    )
    def permute_kernel(key_hbm, idx_hbm, prefix_hbm, tot_hbm,
                       kc0_hbm, kc1_hbm, ic0_hbm, ic1_hbm,
                       val_v, posa_v, posb_v, tot_v, col_v, ctr_v,
                       sval, sem):
        c = lax.axis_index("c")
        s = lax.axis_index("s")
        w = s * NC + c
        dl16 = lax.iota(jnp.int32, L)

        def zero_buf(buf):
            def zl(i, _):
                buf[pl.ds(i * L, L)] = _zero16()
                return 0
            lax.fori_loop(0, HVSEG, zl, 0)

        def zero_spmem(src_buf):
            for q in range(ZONE // HSEG):
                pltpu.sync_copy(src_buf,
                                sval.at[pl.ds(s * ZONE + q * HSEG, HSEG)])

        pltpu.sync_copy(tot_hbm, tot_v)

        def mkidx(i, _):
            ctr_v[pl.ds(i * L, L)] = (i * L + lax.iota(jnp.int32, L)) * NW + w
            return 0
        lax.fori_loop(0, r // L, mkidx, 0)
        pltpu.async_copy(prefix_hbm.at[ctr_v], col_v, sem).wait()

        def scan_step(i, carry):
            t = tot_v[pl.ds(i * L, L)]
            inc = plsc.cumsum(t)
            col_v[pl.ds(i * L, L)] = inc - t + carry + col_v[pl.ds(i * L, L)]
            ctr_v[pl.ds(i * L, L)] = col_v[pl.ds(i * L, L)]
            return carry + jnp.sum(t)
        lax.fori_loop(0, r // L, scan_step, np.int32(0))

        zero_buf(posa_v)
        zero_spmem(posa_v)
        plsc.subcore_barrier()

        for h, pos_v in ((0, posa_v), (1, posb_v)):
            pltpu.sync_copy(key_hbm.at[pl.ds(w * SEG + h * HSEG, HSEG)],
                            val_v)

            def body(i, _):
                k = val_v[pl.ds(i * L, L)]
                if first_pass:
                    k = _to_key(k)
                    val_v[pl.ds(i * L, L)] = k
                d = _digit(k, shift, mask)
                cnt, last = plsc.scan_count(d)
                cur = plsc.load_gather(ctr_v, [d])
                pos_v[pl.ds(i * L, L)] = cur + cnt - 1
                plsc.addupdate_scatter(ctr_v, [d], cnt, mask=last)
                return 0
            lax.fori_loop(0, HVSEG, body, 0)
            pltpu.sync_copy(val_v, sval.at[pos_v])

        plsc.subcore_barrier()

        @pl.when(c == 0)
        def _dk0():
            pltpu.sync_copy(sval.at[pl.ds(s * ZONE, ZONE)],
                            kc0_hbm.at[pl.ds(s * ZONE, ZONE)])

        @pl.when(c == 1)
        def _dk1():
            pltpu.sync_copy(sval.at[pl.ds(s * ZONE, ZONE)],
                            kc1_hbm.at[pl.ds(s * ZONE, ZONE)])

        plsc.subcore_barrier()

        zero_buf(val_v)
        zero_spmem(val_v)
        plsc.subcore_barrier()

        for h, pos_v in ((0, posa_v), (1, posb_v)):
            if first_pass:
                def gen(i, _):
                    val_v[pl.ds(i * L, L)] = (
                        w * SEG + h * HSEG + i * L + dl16)
                    return 0
                lax.fori_loop(0, HVSEG, gen, 0)
            else:
                pltpu.sync_copy(idx_hbm.at[pl.ds(w * SEG + h * HSEG, HSEG)],
                                val_v)
            pltpu.sync_copy(val_v, sval.at[pos_v])

        plsc.subcore_barrier()

        @pl.when(c == 0)
        def _di0():
            pltpu.sync_copy(sval.at[pl.ds(s * ZONE, ZONE)],
                            ic0_hbm.at[pl.ds(s * ZONE, ZONE)])

        @pl.when(c == 1)
        def _di1():
            pltpu.sync_copy(sval.at[pl.ds(s * ZONE, ZONE)],
                            ic1_hbm.at[pl.ds(s * ZONE, ZONE)])

    return permute_kernel


@functools.partial(
    pl.kernel,
    out_type=(
        jax.ShapeDtypeStruct((NP,), jnp.int32),
        jax.ShapeDtypeStruct((NP,), jnp.int32),
    ),
    mesh=_MESH,
    scratch_types=(
        pltpu.VMEM((SEG,), jnp.int32),
        pltpu.VMEM((SEG,), jnp.int32),
        pltpu.VMEM((SEG,), jnp.int32),
        pltpu.VMEM((SEG,), jnp.int32),
        pltpu.SemaphoreType.DMA,
    ),
    compiler_params=_SC_PARAMS,
)
def _merge_kernel(kc0_hbm, kc1_hbm, ic0_hbm, ic1_hbm, key_hbm, idx_hbm,
                  a_v, b_v, e_v, f_v, sem):
    w = _wid()
    off = w * SEG
    pltpu.sync_copy(kc0_hbm.at[pl.ds(off, SEG)], a_v)
    pltpu.sync_copy(kc1_hbm.at[pl.ds(off, SEG)], b_v)
    pltpu.sync_copy(ic0_hbm.at[pl.ds(off, SEG)], e_v)
    pltpu.sync_copy(ic1_hbm.at[pl.ds(off, SEG)], f_v)

    def body(i, _):
        sl = pl.ds(i * L, L)
        a_v[sl] = a_v[sl] + b_v[sl]
        e_v[sl] = e_v[sl] + f_v[sl]
        return 0
    lax.fori_loop(0, VSEG, body, 0)
    pltpu.sync_copy(a_v, key_hbm.at[pl.ds(off, SEG)])
    pltpu.sync_copy(e_v, idx_hbm.at[pl.ds(off, SEG)])


CHUNK = 2048
NFULL = N // CHUNK
TAIL = N - NFULL * CHUNK
JMAX = (NFULL + NW) // NW


@functools.partial(
    pl.kernel,
    out_type=(
        jax.ShapeDtypeStruct((N, D), jnp.float32),
        jax.ShapeDtypeStruct((N,), jnp.float32),
    ),
    mesh=_MESH,
    scratch_types=(
        pltpu.VMEM((CHUNK,), jnp.int32),
        pltpu.VMEM((CHUNK,), jnp.int32),
        pltpu.VMEM((CHUNK, D), jnp.float32),
        pltpu.VMEM((CHUNK,), jnp.float32),
        pltpu.SemaphoreType.DMA,
    ),
    compiler_params=_SC_PARAMS,
)
def _gather_kernel(x_hbm, key_hbm, idx_hbm, xs_hbm, fs_hbm,
                   key_v, idx_v, rows_v, fit_v, sem):
    w = _wid()
    for j in range(JMAX):
        c = w + j * NW
        off = c * CHUNK

        def do(n):
            pltpu.sync_copy(idx_hbm.at[pl.ds(off, n)],
                            idx_v.at[pl.ds(0, n)])
            pltpu.sync_copy(key_hbm.at[pl.ds(off, n)],
                            key_v.at[pl.ds(0, n)])
            pltpu.async_copy(x_hbm.at[idx_v.at[pl.ds(0, n)]],
                             rows_v.at[pl.ds(0, n)], sem).wait()
            pltpu.sync_copy(rows_v.at[pl.ds(0, n)],
                            xs_hbm.at[pl.ds(off, n)])

            def unkey(i, _):
                fit_v[pl.ds(i * L, L)] = _from_key(key_v[pl.ds(i * L, L)])
                return 0
            lax.fori_loop(0, n // L, unkey, 0)
            pltpu.sync_copy(fit_v.at[pl.ds(0, n)],
                            fs_hbm.at[pl.ds(off, n)])

        @pl.when(c < NFULL)
        def _full():
            do(CHUNK)

        @pl.when(c == NFULL)
        def _tail():
            do(TAIL)


def kernel(x, fitness):
    bits = lax.bitcast_convert_type(fitness, jnp.int32)
    pad = lax.full((NP - N,), np.int32(0x7F800000), jnp.int32)
    k = jnp.concatenate([bits, pad])
    idx = k
    for p in range(3):
        h = _make_hist(SHIFTS[p], BITS[p], p == 0)(k)
        pref, tot = _make_scan(BITS[p])(h)
        kc0, kc1, ic0, ic1 = _make_permute(SHIFTS[p], BITS[p], p == 0)(
            k, idx, pref, tot)
        k, idx = _merge_kernel(kc0, kc1, ic0, ic1)
    x_sorted, fitness_sorted = _gather_kernel(x, k, idx)
    return (x_sorted, fitness_sorted)


# speedup vs baseline: 6.5356x; 1.0720x over previous
import functools

import jax
import jax.numpy as jnp
import numpy as np
from jax import lax
from jax.experimental import pallas as pl
from jax.experimental.pallas import tpu as pltpu
from jax.experimental.pallas import tpu_sc as plsc

N = 1000000
D = 16
NC = 2
NS = 16
NW = NC * NS
L = 16

NP = 1000448
SEG = NP // NW
VSEG = SEG // L
HSEG = SEG // 2
HVSEG = HSEG // L
ZONE = NP // NS

BITS = (11, 11, 10)
SHIFTS = (0, 11, 22)
SIGN = np.int32(-2147483648)

_MESH = plsc.VectorSubcoreMesh(core_axis_name="c", subcore_axis_name="s",
                               num_cores=NC, num_subcores=NS)
_SC_PARAMS = pltpu.CompilerParams(use_tc_tiling_on_sc=False,
                                  needs_layout_passes=False)


def _wid():
    return lax.axis_index("s") * NC + lax.axis_index("c")


def _to_key(b):
    return jnp.where(b < 0, ~b, b | SIGN)


def _from_key(k):
    return plsc.bitcast(jnp.where(k < 0, k ^ SIGN, ~k), jnp.float32)


def _digit(k, shift, mask):
    return lax.shift_right_logical(k, np.int32(shift)) & np.int32(mask)


def _zero16():
    return lax.full((L,), np.int32(0), jnp.int32)


def _make_hist0(shift, nbits):
    r = 1 << nbits
    mask = r - 1

    @functools.partial(
        pl.kernel,
        out_type=jax.ShapeDtypeStruct((NW * r,), jnp.int32),
        mesh=_MESH,
        scratch_types=(
            pltpu.VMEM((SEG,), jnp.int32),
            pltpu.VMEM((r,), jnp.int32),
            pltpu.SemaphoreType.DMA,
        ),
        compiler_params=_SC_PARAMS,
    )
    def hist_kernel(key_hbm, hist_hbm, key_v, hist_v, sem):
        w = _wid()
        for i in range(r // L):
            hist_v[pl.ds(i * L, L)] = _zero16()
        pltpu.sync_copy(key_hbm.at[pl.ds(w * SEG, SEG)], key_v)

        def body(i, _):
            k = _to_key(key_v[pl.ds(i * L, L)])
            d = _digit(k, shift, mask)
            cnt, last = plsc.scan_count(d)
            plsc.addupdate_scatter(hist_v, [d], cnt, mask=last)
            return 0

        lax.fori_loop(0, VSEG, body, 0)
        pltpu.sync_copy(hist_v, hist_hbm.at[pl.ds(w * r, r)])

    return hist_kernel


def _make_merge_hist(shift, nbits):
    r = 1 << nbits
    mask = r - 1

    @functools.partial(
        pl.kernel,
        out_type=(
            jax.ShapeDtypeStruct((NP,), jnp.int32),
            jax.ShapeDtypeStruct((NP,), jnp.int32),
            jax.ShapeDtypeStruct((NW * r,), jnp.int32),
        ),
        mesh=_MESH,
        scratch_types=(
            pltpu.VMEM((SEG,), jnp.int32),
            pltpu.VMEM((SEG,), jnp.int32),
            pltpu.VMEM((r,), jnp.int32),
            pltpu.SemaphoreType.DMA,
        ),
        compiler_params=_SC_PARAMS,
    )
    def mh_kernel(kc0_hbm, kc1_hbm, ic0_hbm, ic1_hbm,
                  key_hbm, idx_hbm, hist_hbm, a_v, b_v, hist_v, sem):
        w = _wid()
        off = w * SEG
        for i in range(r // L):
            hist_v[pl.ds(i * L, L)] = _zero16()
        pltpu.sync_copy(kc0_hbm.at[pl.ds(off, SEG)], a_v)
        pltpu.sync_copy(kc1_hbm.at[pl.ds(off, SEG)], b_v)

        def kbody(i, _):
            sl = pl.ds(i * L, L)
            k = a_v[sl] + b_v[sl]
            a_v[sl] = k
            d = _digit(k, shift, mask)
            cnt, last = plsc.scan_count(d)
            plsc.addupdate_scatter(hist_v, [d], cnt, mask=last)
            return 0
        lax.fori_loop(0, VSEG, kbody, 0)
        pltpu.sync_copy(a_v, key_hbm.at[pl.ds(off, SEG)])
        pltpu.sync_copy(hist_v, hist_hbm.at[pl.ds(w * r, r)])

        pltpu.sync_copy(ic0_hbm.at[pl.ds(off, SEG)], a_v)
        pltpu.sync_copy(ic1_hbm.at[pl.ds(off, SEG)], b_v)

        def ibody(i, _):
            sl = pl.ds(i * L, L)
            a_v[sl] = a_v[sl] + b_v[sl]
            return 0
        lax.fori_loop(0, VSEG, ibody, 0)
        pltpu.sync_copy(a_v, idx_hbm.at[pl.ds(off, SEG)])

    return mh_kernel


def _make_permute(shift, nbits, first_pass):
    r = 1 << nbits
    mask = r - 1

    scratch = (
        pltpu.VMEM((HSEG,), jnp.int32),
        pltpu.VMEM((HSEG,), jnp.int32),
        pltpu.VMEM((HSEG,), jnp.int32),
        pltpu.VMEM((r,), jnp.int32),
        pltpu.VMEM((r,), jnp.int32),
        pltpu.VMEM((r,), jnp.int32),
        pltpu.VMEM((r,), jnp.int32),
        pltpu.VMEM((r,), jnp.int32),
        pltpu.VMEM_SHARED((NP,), jnp.int32),
        pltpu.SemaphoreType.DMA,
    )
    out_type = (
        jax.ShapeDtypeStruct((NP,), jnp.int32),
        jax.ShapeDtypeStruct((NP,), jnp.int32),
        jax.ShapeDtypeStruct((NP,), jnp.int32),
        jax.ShapeDtypeStruct((NP,), jnp.int32),
    )

    @functools.partial(
        pl.kernel, out_type=out_type, mesh=_MESH,
        scratch_types=scratch, compiler_params=_SC_PARAMS,
    )
    def permute_kernel(key_hbm, idx_hbm, hist_hbm,
                       kc0_hbm, kc1_hbm, ic0_hbm, ic1_hbm,
                       val_v, posa_v, posb_v, tot_v, col_v, ctr_v,
                       hb0_v, hb1_v, sval, sem):
        c = lax.axis_index("c")
        s = lax.axis_index("s")
        w = s * NC + c
        dl16 = lax.iota(jnp.int32, L)

        for i in range(r // L):
            tot_v[pl.ds(i * L, L)] = _zero16()
        bufs = (hb0_v, hb1_v)
        cp0 = pltpu.make_async_copy(hist_hbm.at[pl.ds(0, r)], bufs[0], sem)
        cp0.start()
        for wp in range(NW):
            if wp + 1 < NW:
                nxt = pltpu.make_async_copy(
                    hist_hbm.at[pl.ds((wp + 1) * r, r)],
                    bufs[(wp + 1) % 2], sem)
                nxt.start()
            pltpu.make_async_copy(hist_hbm.at[pl.ds(wp * r, r)],
                                  bufs[wp % 2], sem).wait()

            @pl.when(np.int32(wp) == w)
            def _snap():
                def cl(i, _):
                    col_v[pl.ds(i * L, L)] = tot_v[pl.ds(i * L, L)]
                    return 0
                lax.fori_loop(0, r // L, cl, 0)

            buf = bufs[wp % 2]

            def al(i, _):
                sl = pl.ds(i * L, L)
                tot_v[sl] = tot_v[sl] + buf[sl]
                return 0
            lax.fori_loop(0, r // L, al, 0)

        def scan_step(i, carry):
            t = tot_v[pl.ds(i * L, L)]
            inc = plsc.cumsum(t)
            ctr_v[pl.ds(i * L, L)] = inc - t + carry + col_v[pl.ds(i * L, L)]
            return carry + jnp.sum(t)
        lax.fori_loop(0, r // L, scan_step, np.int32(0))

        def zl(i, _):
            posa_v[pl.ds(i * L, L)] = _zero16()
            return 0
        lax.fori_loop(0, HVSEG, zl, 0)
        for q in range(ZONE // HSEG):
            pltpu.sync_copy(posa_v,
                            sval.at[pl.ds(s * ZONE + q * HSEG, HSEG)])
        plsc.subcore_barrier()

        for h, pos_v in ((0, posa_v), (1, posb_v)):
            pltpu.sync_copy(key_hbm.at[pl.ds(w * SEG + h * HSEG, HSEG)],
                            val_v)

            def body(i, _):
                k = val_v[pl.ds(i * L, L)]
                if first_pass:
                    k = _to_key(k)
                    val_v[pl.ds(i * L, L)] = k
                d = _digit(k, shift, mask)
                cnt, last = plsc.scan_count(d)
                cur = plsc.load_gather(ctr_v, [d])
                pos_v[pl.ds(i * L, L)] = cur + cnt - 1
                plsc.addupdate_scatter(ctr_v, [d], cnt, mask=last)
                return 0
            lax.fori_loop(0, HVSEG, body, 0)
            pltpu.sync_copy(val_v, sval.at[pos_v])

        plsc.subcore_barrier()

        @pl.when(c == 0)
        def _dk0():
            pltpu.sync_copy(sval.at[pl.ds(s * ZONE, ZONE)],
                            kc0_hbm.at[pl.ds(s * ZONE, ZONE)])

        @pl.when(c == 1)
        def _dk1():
            pltpu.sync_copy(sval.at[pl.ds(s * ZONE, ZONE)],
                            kc1_hbm.at[pl.ds(s * ZONE, ZONE)])

        plsc.subcore_barrier()

        for h, pos_v in ((0, posa_v), (1, posb_v)):
            if first_pass:
                def gen(i, _):
                    val_v[pl.ds(i * L, L)] = (
                        w * SEG + h * HSEG + i * L + dl16)
                    return 0
                lax.fori_loop(0, HVSEG, gen, 0)
            else:
                pltpu.sync_copy(idx_hbm.at[pl.ds(w * SEG + h * HSEG, HSEG)],
                                val_v)
            pltpu.sync_copy(val_v, sval.at[pos_v])

        plsc.subcore_barrier()

        @pl.when(c == 0)
        def _di0():
            pltpu.sync_copy(sval.at[pl.ds(s * ZONE, ZONE)],
                            ic0_hbm.at[pl.ds(s * ZONE, ZONE)])

        @pl.when(c == 1)
        def _di1():
            pltpu.sync_copy(sval.at[pl.ds(s * ZONE, ZONE)],
                            ic1_hbm.at[pl.ds(s * ZONE, ZONE)])

    return permute_kernel


CHUNK = 2048
NFULL = N // CHUNK
TAIL = N - NFULL * CHUNK
JMAX = (NFULL + NW) // NW


@functools.partial(
    pl.kernel,
    out_type=(
        jax.ShapeDtypeStruct((N, D), jnp.float32),
        jax.ShapeDtypeStruct((N,), jnp.float32),
    ),
    mesh=_MESH,
    scratch_types=(
        pltpu.VMEM((CHUNK,), jnp.int32),
        pltpu.VMEM((CHUNK,), jnp.int32),
        pltpu.VMEM((CHUNK,), jnp.int32),
        pltpu.VMEM((CHUNK,), jnp.int32),
        pltpu.VMEM((CHUNK, D), jnp.float32),
        pltpu.VMEM((CHUNK,), jnp.float32),
        pltpu.SemaphoreType.DMA,
    ),
    compiler_params=_SC_PARAMS,
)
def _gather_kernel(x_hbm, kc0_hbm, kc1_hbm, ic0_hbm, ic1_hbm,
                   xs_hbm, fs_hbm,
                   k0_v, k1_v, i0_v, i1_v, rows_v, fit_v, sem):
    w = _wid()
    for j in range(JMAX):
        c = w + j * NW
        off = c * CHUNK

        def do(n):
            pltpu.sync_copy(ic0_hbm.at[pl.ds(off, n)], i0_v.at[pl.ds(0, n)])
            pltpu.sync_copy(ic1_hbm.at[pl.ds(off, n)], i1_v.at[pl.ds(0, n)])
            pltpu.sync_copy(kc0_hbm.at[pl.ds(off, n)], k0_v.at[pl.ds(0, n)])
            pltpu.sync_copy(kc1_hbm.at[pl.ds(off, n)], k1_v.at[pl.ds(0, n)])

            def mrg(i, _):
                sl = pl.ds(i * L, L)
                i0_v[sl] = i0_v[sl] + i1_v[sl]
                fit_v[sl] = _from_key(k0_v[sl] + k1_v[sl])
                return 0
            lax.fori_loop(0, n // L, mrg, 0)

            pltpu.async_copy(x_hbm.at[i0_v.at[pl.ds(0, n)]],
                             rows_v.at[pl.ds(0, n)], sem).wait()
            pltpu.sync_copy(rows_v.at[pl.ds(0, n)],
                            xs_hbm.at[pl.ds(off, n)])
            pltpu.sync_copy(fit_v.at[pl.ds(0, n)],
                            fs_hbm.at[pl.ds(off, n)])

        @pl.when(c < NFULL)
        def _full():
            do(CHUNK)

        @pl.when(c == NFULL)
        def _tail():
            do(TAIL)


def kernel(x, fitness):
    bits = lax.bitcast_convert_type(fitness, jnp.int32)
    pad = lax.full((NP - N,), np.int32(0x7F800000), jnp.int32)
    kb = jnp.concatenate([bits, pad])
    h = _make_hist0(SHIFTS[0], BITS[0])(kb)
    kc0, kc1, ic0, ic1 = _make_permute(SHIFTS[0], BITS[0], True)(kb, kb, h)
    for p in (1, 2):
        k, idx, h = _make_merge_hist(SHIFTS[p], BITS[p])(kc0, kc1, ic0, ic1)
        kc0, kc1, ic0, ic1 = _make_permute(SHIFTS[p], BITS[p], False)(
            k, idx, h)
    x_sorted, fitness_sorted = _gather_kernel(x, kc0, kc1, ic0, ic1)
    return (x_sorted, fitness_sorted)


# speedup vs baseline: 6.7338x; 1.0303x over previous
import functools

import jax
import jax.numpy as jnp
import numpy as np
from jax import lax
from jax.experimental import pallas as pl
from jax.experimental.pallas import tpu as pltpu
from jax.experimental.pallas import tpu_sc as plsc

N = 1000000
D = 16
NC = 2
NS = 16
NW = NC * NS
L = 16

NP = 1000448
SEG = NP // NW
VSEG = SEG // L
HSEG = SEG // 2
HVSEG = HSEG // L
ZONE = NP // NS

BITS = (11, 11, 10)
SHIFTS = (0, 11, 22)
SIGN = np.int32(-2147483648)

_MESH = plsc.VectorSubcoreMesh(core_axis_name="c", subcore_axis_name="s",
                               num_cores=NC, num_subcores=NS)
_SC_PARAMS = pltpu.CompilerParams(use_tc_tiling_on_sc=False,
                                  needs_layout_passes=False)


def _wid():
    return lax.axis_index("s") * NC + lax.axis_index("c")


def _to_key(b):
    return jnp.where(b < 0, ~b, b | SIGN)


def _from_key(k):
    return plsc.bitcast(jnp.where(k < 0, k ^ SIGN, ~k), jnp.float32)


def _digit(k, shift, mask):
    return lax.shift_right_logical(k, np.int32(shift)) & np.int32(mask)


def _zero16():
    return lax.full((L,), np.int32(0), jnp.int32)


def _make_hist0(shift, nbits):
    r = 1 << nbits
    mask = r - 1

    @functools.partial(
        pl.kernel,
        out_type=jax.ShapeDtypeStruct((NW * r,), jnp.int32),
        mesh=_MESH,
        scratch_types=(
            pltpu.VMEM((SEG,), jnp.int32),
            pltpu.VMEM((r,), jnp.int32),
            pltpu.SemaphoreType.DMA,
        ),
        compiler_params=_SC_PARAMS,
    )
    def hist_kernel(key_hbm, hist_hbm, key_v, hist_v, sem):
        w = _wid()
        for i in range(r // L):
            hist_v[pl.ds(i * L, L)] = _zero16()
        pltpu.sync_copy(key_hbm.at[pl.ds(w * SEG, SEG)], key_v)

        def body(i, _):
            k = _to_key(key_v[pl.ds(i * L, L)])
            d = _digit(k, shift, mask)
            cnt, last = plsc.scan_count(d)
            plsc.addupdate_scatter(hist_v, [d], cnt, mask=last)
            return 0

        lax.fori_loop(0, VSEG, body, 0)
        pltpu.sync_copy(hist_v, hist_hbm.at[pl.ds(w * r, r)])

    return hist_kernel


def _make_merge_hist(shift, nbits):
    r = 1 << nbits
    mask = r - 1

    @functools.partial(
        pl.kernel,
        out_type=(
            jax.ShapeDtypeStruct((NP,), jnp.int32),
            jax.ShapeDtypeStruct((NP,), jnp.int32),
            jax.ShapeDtypeStruct((NW * r,), jnp.int32),
        ),
        mesh=_MESH,
        scratch_types=(
            pltpu.VMEM((SEG,), jnp.int32),
            pltpu.VMEM((SEG,), jnp.int32),
            pltpu.VMEM((r,), jnp.int32),
            pltpu.SemaphoreType.DMA,
        ),
        compiler_params=_SC_PARAMS,
    )
    def mh_kernel(kc0_hbm, kc1_hbm, ic0_hbm, ic1_hbm,
                  key_hbm, idx_hbm, hist_hbm, a_v, b_v, hist_v, sem):
        w = _wid()
        off = w * SEG
        for i in range(r // L):
            hist_v[pl.ds(i * L, L)] = _zero16()
        pltpu.sync_copy(kc0_hbm.at[pl.ds(off, SEG)], a_v)
        pltpu.sync_copy(kc1_hbm.at[pl.ds(off, SEG)], b_v)

        def kbody(i, _):
            sl = pl.ds(i * L, L)
            k = a_v[sl] + b_v[sl]
            a_v[sl] = k
            d = _digit(k, shift, mask)
            cnt, last = plsc.scan_count(d)
            plsc.addupdate_scatter(hist_v, [d], cnt, mask=last)
            return 0
        lax.fori_loop(0, VSEG, kbody, 0)
        pltpu.sync_copy(a_v, key_hbm.at[pl.ds(off, SEG)])
        pltpu.sync_copy(hist_v, hist_hbm.at[pl.ds(w * r, r)])

        pltpu.sync_copy(ic0_hbm.at[pl.ds(off, SEG)], a_v)
        pltpu.sync_copy(ic1_hbm.at[pl.ds(off, SEG)], b_v)

        def ibody(i, _):
            sl = pl.ds(i * L, L)
            a_v[sl] = a_v[sl] + b_v[sl]
            return 0
        lax.fori_loop(0, VSEG, ibody, 0)
        pltpu.sync_copy(a_v, idx_hbm.at[pl.ds(off, SEG)])

    return mh_kernel


def _make_permute(shift, nbits, first_pass):
    r = 1 << nbits
    mask = r - 1

    scratch = (
        pltpu.VMEM((HSEG,), jnp.int32),
        pltpu.VMEM((HSEG,), jnp.int32),
        pltpu.VMEM((HSEG,), jnp.int32),
        pltpu.VMEM((r,), jnp.int32),
        pltpu.VMEM((r,), jnp.int32),
        pltpu.VMEM((r,), jnp.int32),
        pltpu.VMEM((r,), jnp.int32),
        pltpu.VMEM((r,), jnp.int32),
        pltpu.VMEM_SHARED((NP,), jnp.int32),
        pltpu.SemaphoreType.DMA,
    )
    out_type = (
        jax.ShapeDtypeStruct((NP,), jnp.int32),
        jax.ShapeDtypeStruct((NP,), jnp.int32),
        jax.ShapeDtypeStruct((NP,), jnp.int32),
        jax.ShapeDtypeStruct((NP,), jnp.int32),
    )

    @functools.partial(
        pl.kernel, out_type=out_type, mesh=_MESH,
        scratch_types=scratch, compiler_params=_SC_PARAMS,
    )
    def permute_kernel(key_hbm, idx_hbm, hist_hbm,
                       kc0_hbm, kc1_hbm, ic0_hbm, ic1_hbm,
                       val_v, posa_v, posb_v, tot_v, col_v, ctr_v,
                       hb0_v, hb1_v, sval, sem):
        c = lax.axis_index("c")
        s = lax.axis_index("s")
        w = s * NC + c
        dl16 = lax.iota(jnp.int32, L)

        for i in range(r // L):
            tot_v[pl.ds(i * L, L)] = _zero16()
        bufs = (hb0_v, hb1_v)
        cp0 = pltpu.make_async_copy(hist_hbm.at[pl.ds(0, r)], bufs[0], sem)
        cp0.start()
        for wp in range(NW):
            if wp + 1 < NW:
                nxt = pltpu.make_async_copy(
                    hist_hbm.at[pl.ds((wp + 1) * r, r)],
                    bufs[(wp + 1) % 2], sem)
                nxt.start()
            pltpu.make_async_copy(hist_hbm.at[pl.ds(wp * r, r)],
                                  bufs[wp % 2], sem).wait()

            @pl.when(np.int32(wp) == w)
            def _snap():
                def cl(i, _):
                    col_v[pl.ds(i * L, L)] = tot_v[pl.ds(i * L, L)]
                    return 0
                lax.fori_loop(0, r // L, cl, 0)

            buf = bufs[wp % 2]

            def al(i, _):
                sl = pl.ds(i * L, L)
                tot_v[sl] = tot_v[sl] + buf[sl]
                return 0
            lax.fori_loop(0, r // L, al, 0)

        def scan_step(i, carry):
            t = tot_v[pl.ds(i * L, L)]
            inc = plsc.cumsum(t)
            ctr_v[pl.ds(i * L, L)] = inc - t + carry + col_v[pl.ds(i * L, L)]
            return carry + jnp.sum(t)
        lax.fori_loop(0, r // L, scan_step, np.int32(0))

        def zl(i, _):
            posa_v[pl.ds(i * L, L)] = _zero16()
            return 0
        lax.fori_loop(0, HVSEG, zl, 0)
        for q in range(ZONE // HSEG):
            pltpu.sync_copy(posa_v,
                            sval.at[pl.ds(s * ZONE + q * HSEG, HSEG)])
        plsc.subcore_barrier()

        for h, pos_v in ((0, posa_v), (1, posb_v)):
            pltpu.sync_copy(key_hbm.at[pl.ds(w * SEG + h * HSEG, HSEG)],
                            val_v)

            def body(i, _):
                k = val_v[pl.ds(i * L, L)]
                if first_pass:
                    k = _to_key(k)
                    val_v[pl.ds(i * L, L)] = k
                d = _digit(k, shift, mask)
                cnt, last = plsc.scan_count(d)
                cur = plsc.load_gather(ctr_v, [d])
                pos_v[pl.ds(i * L, L)] = cur + cnt - 1
                plsc.addupdate_scatter(ctr_v, [d], cnt, mask=last)
                return 0
            lax.fori_loop(0, HVSEG, body, 0)
            pltpu.sync_copy(val_v, sval.at[pos_v])

        plsc.subcore_barrier()

        @pl.when(c == 0)
        def _dk0():
            pltpu.sync_copy(sval.at[pl.ds(s * ZONE, ZONE)],
                            kc0_hbm.at[pl.ds(s * ZONE, ZONE)])

        @pl.when(c == 1)
        def _dk1():
            pltpu.sync_copy(sval.at[pl.ds(s * ZONE, ZONE)],
                            kc1_hbm.at[pl.ds(s * ZONE, ZONE)])

        plsc.subcore_barrier()

        for h, pos_v in ((0, posa_v), (1, posb_v)):
            if first_pass:
                def gen(i, _):
                    val_v[pl.ds(i * L, L)] = (
                        w * SEG + h * HSEG + i * L + dl16)
                    return 0
                lax.fori_loop(0, HVSEG, gen, 0)
            else:
                pltpu.sync_copy(idx_hbm.at[pl.ds(w * SEG + h * HSEG, HSEG)],
                                val_v)
            pltpu.sync_copy(val_v, sval.at[pos_v])

        plsc.subcore_barrier()

        @pl.when(c == 0)
        def _di0():
            pltpu.sync_copy(sval.at[pl.ds(s * ZONE, ZONE)],
                            ic0_hbm.at[pl.ds(s * ZONE, ZONE)])

        @pl.when(c == 1)
        def _di1():
            pltpu.sync_copy(sval.at[pl.ds(s * ZONE, ZONE)],
                            ic1_hbm.at[pl.ds(s * ZONE, ZONE)])

    return permute_kernel


CHUNK = 2048
NFULL = N // CHUNK
TAIL = N - NFULL * CHUNK
NJ = 15
NEXTRA = NFULL - NJ * NW


@functools.partial(
    pl.kernel,
    out_type=(
        jax.ShapeDtypeStruct((N, D), jnp.float32),
        jax.ShapeDtypeStruct((N,), jnp.float32),
    ),
    mesh=_MESH,
    scratch_types=(
        pltpu.VMEM((2, CHUNK), jnp.int32),
        pltpu.VMEM((2, CHUNK), jnp.int32),
        pltpu.VMEM((2, CHUNK), jnp.int32),
        pltpu.VMEM((2, CHUNK), jnp.int32),
        pltpu.VMEM((2, CHUNK, D), jnp.float32),
        pltpu.VMEM((2, CHUNK), jnp.float32),
        pltpu.SemaphoreType.DMA,
        pltpu.SemaphoreType.DMA,
        pltpu.SemaphoreType.DMA,
    ),
    compiler_params=_SC_PARAMS,
)
def _gather_kernel(x_hbm, kc0_hbm, kc1_hbm, ic0_hbm, ic1_hbm,
                   xs_hbm, fs_hbm,
                   i0_v, i1_v, k0_v, k1_v, rows_v, fit_v,
                   sem_in, sem_row, sem_out):
    w = _wid()

    def start_in(j, p):
        off = (w + j * NW) * CHUNK
        return [
            pltpu.async_copy(ic0_hbm.at[pl.ds(off, CHUNK)], i0_v.at[p],
                             sem_in),
            pltpu.async_copy(ic1_hbm.at[pl.ds(off, CHUNK)], i1_v.at[p],
                             sem_in),
            pltpu.async_copy(kc0_hbm.at[pl.ds(off, CHUNK)], k0_v.at[p],
                             sem_in),
            pltpu.async_copy(kc1_hbm.at[pl.ds(off, CHUNK)], k1_v.at[p],
                             sem_in),
        ]

    pend_in = {0: start_in(0, 0)}
    pend_out = {}
    for j in range(NJ):
        p = j % 2
        off = (w + j * NW) * CHUNK
        if j >= 2:
            for cp in pend_out.pop(j - 2):
                cp.wait()
        if j + 1 < NJ:
            pend_in[j + 1] = start_in(j + 1, 1 - p)
        for cp in pend_in.pop(j):
            cp.wait()

        def mrg(i, _):
            sl = pl.ds(i * L, L)
            i0_v[p, sl] = i0_v[p, sl] + i1_v[p, sl]
            fit_v[p, sl] = _from_key(k0_v[p, sl] + k1_v[p, sl])
            return 0
        lax.fori_loop(0, CHUNK // L, mrg, 0)

        pltpu.async_copy(x_hbm.at[i0_v.at[p]], rows_v.at[p], sem_row).wait()
        pend_out[j] = [
            pltpu.async_copy(rows_v.at[p], xs_hbm.at[pl.ds(off, CHUNK)],
                             sem_out),
            pltpu.async_copy(fit_v.at[p], fs_hbm.at[pl.ds(off, CHUNK)],
                             sem_out),
        ]
    for js in sorted(pend_out):
        for cp in pend_out[js]:
            cp.wait()

    def do(off, n):
        pltpu.sync_copy(ic0_hbm.at[pl.ds(off, n)], i0_v.at[0, pl.ds(0, n)])
        pltpu.sync_copy(ic1_hbm.at[pl.ds(off, n)], i1_v.at[0, pl.ds(0, n)])
        pltpu.sync_copy(kc0_hbm.at[pl.ds(off, n)], k0_v.at[0, pl.ds(0, n)])
        pltpu.sync_copy(kc1_hbm.at[pl.ds(off, n)], k1_v.at[0, pl.ds(0, n)])

        def mrg(i, _):
            sl = pl.ds(i * L, L)
            i0_v[0, sl] = i0_v[0, sl] + i1_v[0, sl]
            fit_v[0, sl] = _from_key(k0_v[0, sl] + k1_v[0, sl])
            return 0
        lax.fori_loop(0, n // L, mrg, 0)
        pltpu.async_copy(x_hbm.at[i0_v.at[0, pl.ds(0, n)]],
                         rows_v.at[0, pl.ds(0, n)], sem_row).wait()
        pltpu.sync_copy(rows_v.at[0, pl.ds(0, n)],
                        xs_hbm.at[pl.ds(off, n)])
        pltpu.sync_copy(fit_v.at[0, pl.ds(0, n)],
                        fs_hbm.at[pl.ds(off, n)])

    @pl.when(w < NEXTRA)
    def _extra():
        do((NJ * NW + w) * CHUNK, CHUNK)

    @pl.when(w == NEXTRA)
    def _tail():
        do(NFULL * CHUNK, TAIL)


def kernel(x, fitness):
    bits = lax.bitcast_convert_type(fitness, jnp.int32)
    pad = lax.full((NP - N,), np.int32(0x7F800000), jnp.int32)
    kb = jnp.concatenate([bits, pad])
    h = _make_hist0(SHIFTS[0], BITS[0])(kb)
    kc0, kc1, ic0, ic1 = _make_permute(SHIFTS[0], BITS[0], True)(kb, kb, h)
    for p in (1, 2):
        k, idx, h = _make_merge_hist(SHIFTS[p], BITS[p])(kc0, kc1, ic0, ic1)
        kc0, kc1, ic0, ic1 = _make_permute(SHIFTS[p], BITS[p], False)(
            k, idx, h)
    x_sorted, fitness_sorted = _gather_kernel(x, kc0, kc1, ic0, ic1)
    return (x_sorted, fitness_sorted)


# speedup vs baseline: 6.9960x; 1.0389x over previous
import functools

import jax
import jax.numpy as jnp
import numpy as np
from jax import lax
from jax.experimental import pallas as pl
from jax.experimental.pallas import tpu as pltpu
from jax.experimental.pallas import tpu_sc as plsc

N = 1000000
D = 16
NC = 2
NS = 16
NW = NC * NS
L = 16

NP = 1000448
SEG = NP // NW
VSEG = SEG // L
HSEG = SEG // 2
HVSEG = HSEG // L
ZONE = NP // NS

BITS = (11, 11, 10)
SHIFTS = (0, 11, 22)
SIGN = np.int32(-2147483648)

_MESH = plsc.VectorSubcoreMesh(core_axis_name="c", subcore_axis_name="s",
                               num_cores=NC, num_subcores=NS)
_SC_PARAMS = pltpu.CompilerParams(use_tc_tiling_on_sc=False,
                                  needs_layout_passes=False)


def _wid():
    return lax.axis_index("s") * NC + lax.axis_index("c")


def _to_key(b):
    return jnp.where(b < 0, ~b, b | SIGN)


def _from_key(k):
    return plsc.bitcast(jnp.where(k < 0, k ^ SIGN, ~k), jnp.float32)


def _digit(k, shift, mask):
    return lax.shift_right_logical(k, np.int32(shift)) & np.int32(mask)


def _zero16():
    return lax.full((L,), np.int32(0), jnp.int32)


def _make_hist0(shift, nbits):
    r = 1 << nbits
    mask = r - 1

    @functools.partial(
        pl.kernel,
        out_type=jax.ShapeDtypeStruct((NW * r,), jnp.int32),
        mesh=_MESH,
        scratch_types=(
            pltpu.VMEM((SEG,), jnp.int32),
            pltpu.VMEM((r,), jnp.int32),
            pltpu.SemaphoreType.DMA,
        ),
        compiler_params=_SC_PARAMS,
    )
    def hist_kernel(key_hbm, hist_hbm, key_v, hist_v, sem):
        w = _wid()
        for i in range(r // L):
            hist_v[pl.ds(i * L, L)] = _zero16()
        pltpu.sync_copy(key_hbm.at[pl.ds(w * SEG, SEG)], key_v)

        def rank_of(i):
            d = _digit(_to_key(key_v[pl.ds(i * L, L)]), shift, mask)
            cnt, last = plsc.scan_count(d)
            return d, cnt, last

        def body(i, carry):
            d, cnt, last = carry
            nxt = rank_of(i + 1)
            plsc.addupdate_scatter(hist_v, [d], cnt, mask=last)
            return nxt

        last_c = lax.fori_loop(0, VSEG - 1, body, rank_of(0))
        d, cnt, last = last_c
        plsc.addupdate_scatter(hist_v, [d], cnt, mask=last)
        pltpu.sync_copy(hist_v, hist_hbm.at[pl.ds(w * r, r)])

    return hist_kernel


def _make_merge_hist(shift, nbits):
    r = 1 << nbits
    mask = r - 1

    @functools.partial(
        pl.kernel,
        out_type=(
            jax.ShapeDtypeStruct((NP,), jnp.int32),
            jax.ShapeDtypeStruct((NP,), jnp.int32),
            jax.ShapeDtypeStruct((NW * r,), jnp.int32),
        ),
        mesh=_MESH,
        scratch_types=(
            pltpu.VMEM((SEG,), jnp.int32),
            pltpu.VMEM((SEG,), jnp.int32),
            pltpu.VMEM((r,), jnp.int32),
            pltpu.SemaphoreType.DMA,
        ),
        compiler_params=_SC_PARAMS,
    )
    def mh_kernel(kc0_hbm, kc1_hbm, ic0_hbm, ic1_hbm,
                  key_hbm, idx_hbm, hist_hbm, a_v, b_v, hist_v, sem):
        w = _wid()
        off = w * SEG
        for i in range(r // L):
            hist_v[pl.ds(i * L, L)] = _zero16()
        pltpu.sync_copy(kc0_hbm.at[pl.ds(off, SEG)], a_v)
        pltpu.sync_copy(kc1_hbm.at[pl.ds(off, SEG)], b_v)

        def mrg_rank(i):
            sl = pl.ds(i * L, L)
            k = a_v[sl] + b_v[sl]
            a_v[sl] = k
            d = _digit(k, shift, mask)
            cnt, last = plsc.scan_count(d)
            return d, cnt, last

        def kbody(i, carry):
            d, cnt, last = carry
            nxt = mrg_rank(i + 1)
            plsc.addupdate_scatter(hist_v, [d], cnt, mask=last)
            return nxt

        d, cnt, last = lax.fori_loop(0, VSEG - 1, kbody, mrg_rank(0))
        plsc.addupdate_scatter(hist_v, [d], cnt, mask=last)
        pltpu.sync_copy(a_v, key_hbm.at[pl.ds(off, SEG)])
        pltpu.sync_copy(hist_v, hist_hbm.at[pl.ds(w * r, r)])

        pltpu.sync_copy(ic0_hbm.at[pl.ds(off, SEG)], a_v)
        pltpu.sync_copy(ic1_hbm.at[pl.ds(off, SEG)], b_v)

        def ibody(i, _):
            sl = pl.ds(i * L, L)
            a_v[sl] = a_v[sl] + b_v[sl]
            return 0
        lax.fori_loop(0, VSEG, ibody, 0)
        pltpu.sync_copy(a_v, idx_hbm.at[pl.ds(off, SEG)])

    return mh_kernel


def _make_permute(shift, nbits, first_pass):
    r = 1 << nbits
    mask = r - 1

    scratch = (
        pltpu.VMEM((HSEG,), jnp.int32),
        pltpu.VMEM((HSEG,), jnp.int32),
        pltpu.VMEM((HSEG,), jnp.int32),
        pltpu.VMEM((r,), jnp.int32),
        pltpu.VMEM((r,), jnp.int32),
        pltpu.VMEM((r,), jnp.int32),
        pltpu.VMEM((r,), jnp.int32),
        pltpu.VMEM((r,), jnp.int32),
        pltpu.VMEM_SHARED((NP,), jnp.int32),
        pltpu.SemaphoreType.DMA,
    )
    out_type = (
        jax.ShapeDtypeStruct((NP,), jnp.int32),
        jax.ShapeDtypeStruct((NP,), jnp.int32),
        jax.ShapeDtypeStruct((NP,), jnp.int32),
        jax.ShapeDtypeStruct((NP,), jnp.int32),
    )

    @functools.partial(
        pl.kernel, out_type=out_type, mesh=_MESH,
        scratch_types=scratch, compiler_params=_SC_PARAMS,
    )
    def permute_kernel(key_hbm, idx_hbm, hist_hbm,
                       kc0_hbm, kc1_hbm, ic0_hbm, ic1_hbm,
                       val_v, posa_v, posb_v, tot_v, col_v, ctr_v,
                       hb0_v, hb1_v, sval, sem):
        c = lax.axis_index("c")
        s = lax.axis_index("s")
        w = s * NC + c
        dl16 = lax.iota(jnp.int32, L)

        for i in range(r // L):
            tot_v[pl.ds(i * L, L)] = _zero16()
        bufs = (hb0_v, hb1_v)
        cp0 = pltpu.make_async_copy(hist_hbm.at[pl.ds(0, r)], bufs[0], sem)
        cp0.start()
        for wp in range(NW):
            if wp + 1 < NW:
                nxt = pltpu.make_async_copy(
                    hist_hbm.at[pl.ds((wp + 1) * r, r)],
                    bufs[(wp + 1) % 2], sem)
                nxt.start()
            pltpu.make_async_copy(hist_hbm.at[pl.ds(wp * r, r)],
                                  bufs[wp % 2], sem).wait()

            @pl.when(np.int32(wp) == w)
            def _snap():
                def cl(i, _):
                    col_v[pl.ds(i * L, L)] = tot_v[pl.ds(i * L, L)]
                    return 0
                lax.fori_loop(0, r // L, cl, 0)

            buf = bufs[wp % 2]

            def al(i, _):
                sl = pl.ds(i * L, L)
                tot_v[sl] = tot_v[sl] + buf[sl]
                return 0
            lax.fori_loop(0, r // L, al, 0)

        def scan_step(i, carry):
            t = tot_v[pl.ds(i * L, L)]
            inc = plsc.cumsum(t)
            ctr_v[pl.ds(i * L, L)] = inc - t + carry + col_v[pl.ds(i * L, L)]
            return carry + jnp.sum(t)
        lax.fori_loop(0, r // L, scan_step, np.int32(0))

        def zl(i, _):
            posa_v[pl.ds(i * L, L)] = _zero16()
            return 0
        lax.fori_loop(0, HVSEG, zl, 0)
        for q in range(ZONE // HSEG):
            pltpu.sync_copy(posa_v,
                            sval.at[pl.ds(s * ZONE + q * HSEG, HSEG)])
        plsc.subcore_barrier()

        for h, pos_v in ((0, posa_v), (1, posb_v)):
            pltpu.sync_copy(key_hbm.at[pl.ds(w * SEG + h * HSEG, HSEG)],
                            val_v)

            def rank_of(i):
                k = val_v[pl.ds(i * L, L)]
                if first_pass:
                    k = _to_key(k)
                    val_v[pl.ds(i * L, L)] = k
                d = _digit(k, shift, mask)
                cnt, last = plsc.scan_count(d)
                return d, cnt, last

            def place(i, d, cnt, last):
                cur = plsc.load_gather(ctr_v, [d])
                pos_v[pl.ds(i * L, L)] = cur + cnt - 1
                plsc.addupdate_scatter(ctr_v, [d], cnt, mask=last)

            def body(i, carry):
                d, cnt, last = carry
                nxt = rank_of(i + 1)
                place(i, d, cnt, last)
                return nxt

            d, cnt, last = lax.fori_loop(0, HVSEG - 1, body, rank_of(0))
            place(HVSEG - 1, d, cnt, last)
            pltpu.sync_copy(val_v, sval.at[pos_v])

        plsc.subcore_barrier()

        @pl.when(c == 0)
        def _dk0():
            pltpu.sync_copy(sval.at[pl.ds(s * ZONE, ZONE)],
                            kc0_hbm.at[pl.ds(s * ZONE, ZONE)])

        @pl.when(c == 1)
        def _dk1():
            pltpu.sync_copy(sval.at[pl.ds(s * ZONE, ZONE)],
                            kc1_hbm.at[pl.ds(s * ZONE, ZONE)])

        plsc.subcore_barrier()

        for h, pos_v in ((0, posa_v), (1, posb_v)):
            if first_pass:
                def gen(i, _):
                    val_v[pl.ds(i * L, L)] = (
                        w * SEG + h * HSEG + i * L + dl16)
                    return 0
                lax.fori_loop(0, HVSEG, gen, 0)
            else:
                pltpu.sync_copy(idx_hbm.at[pl.ds(w * SEG + h * HSEG, HSEG)],
                                val_v)
            pltpu.sync_copy(val_v, sval.at[pos_v])

        plsc.subcore_barrier()

        @pl.when(c == 0)
        def _di0():
            pltpu.sync_copy(sval.at[pl.ds(s * ZONE, ZONE)],
                            ic0_hbm.at[pl.ds(s * ZONE, ZONE)])

        @pl.when(c == 1)
        def _di1():
            pltpu.sync_copy(sval.at[pl.ds(s * ZONE, ZONE)],
                            ic1_hbm.at[pl.ds(s * ZONE, ZONE)])

    return permute_kernel


CHUNK = 2048
NFULL = N // CHUNK
TAIL = N - NFULL * CHUNK
NJ = 15
NEXTRA = NFULL - NJ * NW


@functools.partial(
    pl.kernel,
    out_type=(
        jax.ShapeDtypeStruct((N, D), jnp.float32),
        jax.ShapeDtypeStruct((N,), jnp.float32),
    ),
    mesh=_MESH,
    scratch_types=(
        pltpu.VMEM((2, CHUNK), jnp.int32),
        pltpu.VMEM((2, CHUNK), jnp.int32),
        pltpu.VMEM((2, CHUNK), jnp.int32),
        pltpu.VMEM((2, CHUNK), jnp.int32),
        pltpu.VMEM((2, CHUNK, D), jnp.float32),
        pltpu.VMEM((2, CHUNK), jnp.float32),
        pltpu.SemaphoreType.DMA,
        pltpu.SemaphoreType.DMA,
        pltpu.SemaphoreType.DMA,
    ),
    compiler_params=_SC_PARAMS,
)
def _gather_kernel(x_hbm, kc0_hbm, kc1_hbm, ic0_hbm, ic1_hbm,
                   xs_hbm, fs_hbm,
                   i0_v, i1_v, k0_v, k1_v, rows_v, fit_v,
                   sem_in, sem_row, sem_out):
    w = _wid()

    def start_in(j, p):
        off = (w + j * NW) * CHUNK
        return [
            pltpu.async_copy(ic0_hbm.at[pl.ds(off, CHUNK)], i0_v.at[p],
                             sem_in),
            pltpu.async_copy(ic1_hbm.at[pl.ds(off, CHUNK)], i1_v.at[p],
                             sem_in),
            pltpu.async_copy(kc0_hbm.at[pl.ds(off, CHUNK)], k0_v.at[p],
                             sem_in),
            pltpu.async_copy(kc1_hbm.at[pl.ds(off, CHUNK)], k1_v.at[p],
                             sem_in),
        ]

    pend_in = {0: start_in(0, 0)}
    pend_out = {}
    for j in range(NJ):
        p = j % 2
        off = (w + j * NW) * CHUNK
        if j >= 2:
            for cp in pend_out.pop(j - 2):
                cp.wait()
        if j + 1 < NJ:
            pend_in[j + 1] = start_in(j + 1, 1 - p)
        for cp in pend_in.pop(j):
            cp.wait()

        def mrg(i, _):
            sl = pl.ds(i * L, L)
            i0_v[p, sl] = i0_v[p, sl] + i1_v[p, sl]
            fit_v[p, sl] = _from_key(k0_v[p, sl] + k1_v[p, sl])
            return 0
        lax.fori_loop(0, CHUNK // L, mrg, 0)

        pltpu.async_copy(x_hbm.at[i0_v.at[p]], rows_v.at[p], sem_row).wait()
        pend_out[j] = [
            pltpu.async_copy(rows_v.at[p], xs_hbm.at[pl.ds(off, CHUNK)],
                             sem_out),
            pltpu.async_copy(fit_v.at[p], fs_hbm.at[pl.ds(off, CHUNK)],
                             sem_out),
        ]
    for js in sorted(pend_out):
        for cp in pend_out[js]:
            cp.wait()

    def do(off, n):
        pltpu.sync_copy(ic0_hbm.at[pl.ds(off, n)], i0_v.at[0, pl.ds(0, n)])
        pltpu.sync_copy(ic1_hbm.at[pl.ds(off, n)], i1_v.at[0, pl.ds(0, n)])
        pltpu.sync_copy(kc0_hbm.at[pl.ds(off, n)], k0_v.at[0, pl.ds(0, n)])
        pltpu.sync_copy(kc1_hbm.at[pl.ds(off, n)], k1_v.at[0, pl.ds(0, n)])

        def mrg(i, _):
            sl = pl.ds(i * L, L)
            i0_v[0, sl] = i0_v[0, sl] + i1_v[0, sl]
            fit_v[0, sl] = _from_key(k0_v[0, sl] + k1_v[0, sl])
            return 0
        lax.fori_loop(0, n // L, mrg, 0)
        pltpu.async_copy(x_hbm.at[i0_v.at[0, pl.ds(0, n)]],
                         rows_v.at[0, pl.ds(0, n)], sem_row).wait()
        pltpu.sync_copy(rows_v.at[0, pl.ds(0, n)],
                        xs_hbm.at[pl.ds(off, n)])
        pltpu.sync_copy(fit_v.at[0, pl.ds(0, n)],
                        fs_hbm.at[pl.ds(off, n)])

    @pl.when(w < NEXTRA)
    def _extra():
        do((NJ * NW + w) * CHUNK, CHUNK)

    @pl.when(w == NEXTRA)
    def _tail():
        do(NFULL * CHUNK, TAIL)


def kernel(x, fitness):
    bits = lax.bitcast_convert_type(fitness, jnp.int32)
    pad = lax.full((NP - N,), np.int32(0x7F800000), jnp.int32)
    kb = jnp.concatenate([bits, pad])
    h = _make_hist0(SHIFTS[0], BITS[0])(kb)
    kc0, kc1, ic0, ic1 = _make_permute(SHIFTS[0], BITS[0], True)(kb, kb, h)
    for p in (1, 2):
        k, idx, h = _make_merge_hist(SHIFTS[p], BITS[p])(kc0, kc1, ic0, ic1)
        kc0, kc1, ic0, ic1 = _make_permute(SHIFTS[p], BITS[p], False)(
            k, idx, h)
    x_sorted, fitness_sorted = _gather_kernel(x, kc0, kc1, ic0, ic1)
    return (x_sorted, fitness_sorted)


# speedup vs baseline: 7.0556x; 1.0085x over previous
import functools

import jax
import jax.numpy as jnp
import numpy as np
from jax import lax
from jax.experimental import pallas as pl
from jax.experimental.pallas import tpu as pltpu
from jax.experimental.pallas import tpu_sc as plsc

N = 1000000
D = 16
NC = 2
NS = 16
NW = NC * NS
L = 16

NP = 1000448
SEG = NP // NW
VSEG = SEG // L
HSEG = SEG // 2
HVSEG = HSEG // L
ZONE = NP // NS

BITS = (11, 11, 10)
SHIFTS = (0, 11, 22)
SIGN = np.int32(-2147483648)

_MESH = plsc.VectorSubcoreMesh(core_axis_name="c", subcore_axis_name="s",
                               num_cores=NC, num_subcores=NS)
_SC_PARAMS = pltpu.CompilerParams(use_tc_tiling_on_sc=False,
                                  needs_layout_passes=False)


def _wid():
    return lax.axis_index("s") * NC + lax.axis_index("c")


def _to_key(b):
    return jnp.where(b < 0, ~b, b | SIGN)


def _from_key(k):
    return plsc.bitcast(jnp.where(k < 0, k ^ SIGN, ~k), jnp.float32)


def _digit(k, shift, mask):
    return lax.shift_right_logical(k, np.int32(shift)) & np.int32(mask)


def _zero16():
    return lax.full((L,), np.int32(0), jnp.int32)


def _make_hist0(shift, nbits):
    r = 1 << nbits
    mask = r - 1

    @functools.partial(
        pl.kernel,
        out_type=jax.ShapeDtypeStruct((NW * r,), jnp.int32),
        mesh=_MESH,
        scratch_types=(
            pltpu.VMEM((SEG,), jnp.int32),
            pltpu.VMEM((r,), jnp.int32),
            pltpu.SemaphoreType.DMA,
        ),
        compiler_params=_SC_PARAMS,
    )
    def hist_kernel(key_hbm, hist_hbm, key_v, hist_v, sem):
        w = _wid()
        for i in range(r // L):
            hist_v[pl.ds(i * L, L)] = _zero16()
        pltpu.sync_copy(key_hbm.at[pl.ds(w * SEG, SEG)], key_v)

        def rank_of(i):
            d = _digit(_to_key(key_v[pl.ds(i * L, L)]), shift, mask)
            cnt, last = plsc.scan_count(d)
            return d, cnt, last

        def body(i, carry):
            d, cnt, last = carry
            nxt = rank_of(i + 1)
            plsc.addupdate_scatter(hist_v, [d], cnt, mask=last)
            return nxt

        last_c = lax.fori_loop(0, VSEG - 1, body, rank_of(0))
        d, cnt, last = last_c
        plsc.addupdate_scatter(hist_v, [d], cnt, mask=last)
        pltpu.sync_copy(hist_v, hist_hbm.at[pl.ds(w * r, r)])

    return hist_kernel


def _make_merge_hist(shift, nbits):
    r = 1 << nbits
    mask = r - 1

    @functools.partial(
        pl.kernel,
        out_type=(
            jax.ShapeDtypeStruct((NP,), jnp.int32),
            jax.ShapeDtypeStruct((NP,), jnp.int32),
            jax.ShapeDtypeStruct((NW * r,), jnp.int32),
        ),
        mesh=_MESH,
        scratch_types=(
            pltpu.VMEM((SEG,), jnp.int32),
            pltpu.VMEM((SEG,), jnp.int32),
            pltpu.VMEM((SEG,), jnp.int32),
            pltpu.VMEM((SEG,), jnp.int32),
            pltpu.VMEM((r,), jnp.int32),
            pltpu.SemaphoreType.DMA,
            pltpu.SemaphoreType.DMA,
        ),
        compiler_params=_SC_PARAMS,
    )
    def mh_kernel(kc0_hbm, kc1_hbm, ic0_hbm, ic1_hbm,
                  key_hbm, idx_hbm, hist_hbm, a_v, b_v, c_v, d_v, hist_v,
                  sem, sem_out):
        w = _wid()
        off = w * SEG
        cpa = pltpu.async_copy(kc0_hbm.at[pl.ds(off, SEG)], a_v, sem)
        cpb = pltpu.async_copy(kc1_hbm.at[pl.ds(off, SEG)], b_v, sem)
        cpc = pltpu.async_copy(ic0_hbm.at[pl.ds(off, SEG)], c_v, sem)
        cpd = pltpu.async_copy(ic1_hbm.at[pl.ds(off, SEG)], d_v, sem)
        for i in range(r // L):
            hist_v[pl.ds(i * L, L)] = _zero16()
        cpa.wait()
        cpb.wait()

        def mrg_rank(i):
            sl = pl.ds(i * L, L)
            k = a_v[sl] + b_v[sl]
            a_v[sl] = k
            d = _digit(k, shift, mask)
            cnt, last = plsc.scan_count(d)
            return d, cnt, last

        def kbody(i, carry):
            d, cnt, last = carry
            nxt = mrg_rank(i + 1)
            plsc.addupdate_scatter(hist_v, [d], cnt, mask=last)
            return nxt

        d, cnt, last = lax.fori_loop(0, VSEG - 1, kbody, mrg_rank(0))
        plsc.addupdate_scatter(hist_v, [d], cnt, mask=last)
        ko = pltpu.async_copy(a_v, key_hbm.at[pl.ds(off, SEG)], sem_out)
        ho = pltpu.async_copy(hist_v, hist_hbm.at[pl.ds(w * r, r)], sem_out)
        cpc.wait()
        cpd.wait()

        def ibody(i, _):
            sl = pl.ds(i * L, L)
            c_v[sl] = c_v[sl] + d_v[sl]
            return 0
        lax.fori_loop(0, VSEG, ibody, 0)
        pltpu.sync_copy(c_v, idx_hbm.at[pl.ds(off, SEG)])
        ko.wait()
        ho.wait()

    return mh_kernel


def _make_permute(shift, nbits, first_pass):
    r = 1 << nbits
    mask = r - 1

    scratch = (
        pltpu.VMEM((HSEG,), jnp.int32),
        pltpu.VMEM((HSEG,), jnp.int32),
        pltpu.VMEM((HSEG,), jnp.int32),
        pltpu.VMEM((r,), jnp.int32),
        pltpu.VMEM((r,), jnp.int32),
        pltpu.VMEM((r,), jnp.int32),
        pltpu.VMEM((r,), jnp.int32),
        pltpu.VMEM((r,), jnp.int32),
        pltpu.VMEM_SHARED((NP,), jnp.int32),
        pltpu.SemaphoreType.DMA,
    )
    out_type = (
        jax.ShapeDtypeStruct((NP,), jnp.int32),
        jax.ShapeDtypeStruct((NP,), jnp.int32),
        jax.ShapeDtypeStruct((NP,), jnp.int32),
        jax.ShapeDtypeStruct((NP,), jnp.int32),
    )

    @functools.partial(
        pl.kernel, out_type=out_type, mesh=_MESH,
        scratch_types=scratch, compiler_params=_SC_PARAMS,
    )
    def permute_kernel(key_hbm, idx_hbm, hist_hbm,
                       kc0_hbm, kc1_hbm, ic0_hbm, ic1_hbm,
                       val_v, posa_v, posb_v, tot_v, col_v, ctr_v,
                       hb0_v, hb1_v, sval, sem):
        c = lax.axis_index("c")
        s = lax.axis_index("s")
        w = s * NC + c
        dl16 = lax.iota(jnp.int32, L)

        for i in range(r // L):
            tot_v[pl.ds(i * L, L)] = _zero16()
        bufs = (hb0_v, hb1_v)
        cp0 = pltpu.make_async_copy(hist_hbm.at[pl.ds(0, r)], bufs[0], sem)
        cp0.start()
        for wp in range(NW):
            if wp + 1 < NW:
                nxt = pltpu.make_async_copy(
                    hist_hbm.at[pl.ds((wp + 1) * r, r)],
                    bufs[(wp + 1) % 2], sem)
                nxt.start()
            pltpu.make_async_copy(hist_hbm.at[pl.ds(wp * r, r)],
                                  bufs[wp % 2], sem).wait()

            @pl.when(np.int32(wp) == w)
            def _snap():
                def cl(i, _):
                    col_v[pl.ds(i * L, L)] = tot_v[pl.ds(i * L, L)]
                    return 0
                lax.fori_loop(0, r // L, cl, 0)

            buf = bufs[wp % 2]

            def al(i, _):
                sl = pl.ds(i * L, L)
                tot_v[sl] = tot_v[sl] + buf[sl]
                return 0
            lax.fori_loop(0, r // L, al, 0)

        def scan_step(i, carry):
            t = tot_v[pl.ds(i * L, L)]
            inc = plsc.cumsum(t)
            ctr_v[pl.ds(i * L, L)] = inc - t + carry + col_v[pl.ds(i * L, L)]
            return carry + jnp.sum(t)
        lax.fori_loop(0, r // L, scan_step, np.int32(0))

        def zl(i, _):
            posa_v[pl.ds(i * L, L)] = _zero16()
            return 0
        lax.fori_loop(0, HVSEG, zl, 0)
        for q in range(ZONE // HSEG):
            pltpu.sync_copy(posa_v,
                            sval.at[pl.ds(s * ZONE + q * HSEG, HSEG)])
        plsc.subcore_barrier()

        for h, pos_v in ((0, posa_v), (1, posb_v)):
            pltpu.sync_copy(key_hbm.at[pl.ds(w * SEG + h * HSEG, HSEG)],
                            val_v)

            def rank_of(i):
                k = val_v[pl.ds(i * L, L)]
                if first_pass:
                    k = _to_key(k)
                    val_v[pl.ds(i * L, L)] = k
                d = _digit(k, shift, mask)
                cnt, last = plsc.scan_count(d)
                return d, cnt, last

            def place(i, d, cnt, last):
                cur = plsc.load_gather(ctr_v, [d])
                pos_v[pl.ds(i * L, L)] = cur + cnt - 1
                plsc.addupdate_scatter(ctr_v, [d], cnt, mask=last)

            def body(i, carry):
                d, cnt, last = carry
                nxt = rank_of(i + 1)
                place(i, d, cnt, last)
                return nxt

            d, cnt, last = lax.fori_loop(0, HVSEG - 1, body, rank_of(0))
            place(HVSEG - 1, d, cnt, last)
            pltpu.sync_copy(val_v, sval.at[pos_v])

        plsc.subcore_barrier()

        @pl.when(c == 0)
        def _dk0():
            pltpu.sync_copy(sval.at[pl.ds(s * ZONE, ZONE)],
                            kc0_hbm.at[pl.ds(s * ZONE, ZONE)])

        @pl.when(c == 1)
        def _dk1():
            pltpu.sync_copy(sval.at[pl.ds(s * ZONE, ZONE)],
                            kc1_hbm.at[pl.ds(s * ZONE, ZONE)])

        plsc.subcore_barrier()

        for h, pos_v in ((0, posa_v), (1, posb_v)):
            if first_pass:
                def gen(i, _):
                    val_v[pl.ds(i * L, L)] = (
                        w * SEG + h * HSEG + i * L + dl16)
                    return 0
                lax.fori_loop(0, HVSEG, gen, 0)
            else:
                pltpu.sync_copy(idx_hbm.at[pl.ds(w * SEG + h * HSEG, HSEG)],
                                val_v)
            pltpu.sync_copy(val_v, sval.at[pos_v])

        plsc.subcore_barrier()

        @pl.when(c == 0)
        def _di0():
            pltpu.sync_copy(sval.at[pl.ds(s * ZONE, ZONE)],
                            ic0_hbm.at[pl.ds(s * ZONE, ZONE)])

        @pl.when(c == 1)
        def _di1():
            pltpu.sync_copy(sval.at[pl.ds(s * ZONE, ZONE)],
                            ic1_hbm.at[pl.ds(s * ZONE, ZONE)])

    return permute_kernel


CHUNK = 2048
NFULL = N // CHUNK
TAIL = N - NFULL * CHUNK
NJ = 15
NEXTRA = NFULL - NJ * NW


@functools.partial(
    pl.kernel,
    out_type=(
        jax.ShapeDtypeStruct((N, D), jnp.float32),
        jax.ShapeDtypeStruct((N,), jnp.float32),
    ),
    mesh=_MESH,
    scratch_types=(
        pltpu.VMEM((2, CHUNK), jnp.int32),
        pltpu.VMEM((2, CHUNK), jnp.int32),
        pltpu.VMEM((2, CHUNK), jnp.int32),
        pltpu.VMEM((2, CHUNK), jnp.int32),
        pltpu.VMEM((2, CHUNK, D), jnp.float32),
        pltpu.VMEM((2, CHUNK), jnp.float32),
        pltpu.SemaphoreType.DMA,
        pltpu.SemaphoreType.DMA,
        pltpu.SemaphoreType.DMA,
    ),
    compiler_params=_SC_PARAMS,
)
def _gather_kernel(x_hbm, kc0_hbm, kc1_hbm, ic0_hbm, ic1_hbm,
                   xs_hbm, fs_hbm,
                   i0_v, i1_v, k0_v, k1_v, rows_v, fit_v,
                   sem_in, sem_row, sem_out):
    w = _wid()

    def start_in(j, p):
        off = (w + j * NW) * CHUNK
        return [
            pltpu.async_copy(ic0_hbm.at[pl.ds(off, CHUNK)], i0_v.at[p],
                             sem_in),
            pltpu.async_copy(ic1_hbm.at[pl.ds(off, CHUNK)], i1_v.at[p],
                             sem_in),
            pltpu.async_copy(kc0_hbm.at[pl.ds(off, CHUNK)], k0_v.at[p],
                             sem_in),
            pltpu.async_copy(kc1_hbm.at[pl.ds(off, CHUNK)], k1_v.at[p],
                             sem_in),
        ]

    pend_in = {0: start_in(0, 0)}
    pend_out = {}
    for j in range(NJ):
        p = j % 2
        off = (w + j * NW) * CHUNK
        if j >= 2:
            for cp in pend_out.pop(j - 2):
                cp.wait()
        if j + 1 < NJ:
            pend_in[j + 1] = start_in(j + 1, 1 - p)
        for cp in pend_in.pop(j):
            cp.wait()

        def mrg(i, _):
            sl = pl.ds(i * L, L)
            i0_v[p, sl] = i0_v[p, sl] + i1_v[p, sl]
            fit_v[p, sl] = _from_key(k0_v[p, sl] + k1_v[p, sl])
            return 0
        lax.fori_loop(0, CHUNK // L, mrg, 0)

        pltpu.async_copy(x_hbm.at[i0_v.at[p]], rows_v.at[p], sem_row).wait()
        pend_out[j] = [
            pltpu.async_copy(rows_v.at[p], xs_hbm.at[pl.ds(off, CHUNK)],
                             sem_out),
            pltpu.async_copy(fit_v.at[p], fs_hbm.at[pl.ds(off, CHUNK)],
                             sem_out),
        ]
    for js in sorted(pend_out):
        for cp in pend_out[js]:
            cp.wait()

    def do(off, n):
        pltpu.sync_copy(ic0_hbm.at[pl.ds(off, n)], i0_v.at[0, pl.ds(0, n)])
        pltpu.sync_copy(ic1_hbm.at[pl.ds(off, n)], i1_v.at[0, pl.ds(0, n)])
        pltpu.sync_copy(kc0_hbm.at[pl.ds(off, n)], k0_v.at[0, pl.ds(0, n)])
        pltpu.sync_copy(kc1_hbm.at[pl.ds(off, n)], k1_v.at[0, pl.ds(0, n)])

        def mrg(i, _):
            sl = pl.ds(i * L, L)
            i0_v[0, sl] = i0_v[0, sl] + i1_v[0, sl]
            fit_v[0, sl] = _from_key(k0_v[0, sl] + k1_v[0, sl])
            return 0
        lax.fori_loop(0, n // L, mrg, 0)
        pltpu.async_copy(x_hbm.at[i0_v.at[0, pl.ds(0, n)]],
                         rows_v.at[0, pl.ds(0, n)], sem_row).wait()
        pltpu.sync_copy(rows_v.at[0, pl.ds(0, n)],
                        xs_hbm.at[pl.ds(off, n)])
        pltpu.sync_copy(fit_v.at[0, pl.ds(0, n)],
                        fs_hbm.at[pl.ds(off, n)])

    @pl.when(w < NEXTRA)
    def _extra():
        do((NJ * NW + w) * CHUNK, CHUNK)

    @pl.when(w == NEXTRA)
    def _tail():
        do(NFULL * CHUNK, TAIL)


def kernel(x, fitness):
    bits = lax.bitcast_convert_type(fitness, jnp.int32)
    pad = lax.full((NP - N,), np.int32(0x7F800000), jnp.int32)
    kb = jnp.concatenate([bits, pad])
    h = _make_hist0(SHIFTS[0], BITS[0])(kb)
    kc0, kc1, ic0, ic1 = _make_permute(SHIFTS[0], BITS[0], True)(kb, kb, h)
    for p in (1, 2):
        k, idx, h = _make_merge_hist(SHIFTS[p], BITS[p])(kc0, kc1, ic0, ic1)
        kc0, kc1, ic0, ic1 = _make_permute(SHIFTS[p], BITS[p], False)(
            k, idx, h)
    x_sorted, fitness_sorted = _gather_kernel(x, kc0, kc1, ic0, ic1)
    return (x_sorted, fitness_sorted)
